# speedup vs baseline: 1.3488x; 1.3488x over previous
import jax
import jax.numpy as jnp
from jax import lax
from jax.experimental import pallas as pl
from jax.experimental.pallas import tpu as pltpu
from jax.experimental.pallas import tpu_sc as plsc

_NC = 2
_NS = 16
_NW = _NC * _NS

_B = 4
_T = 2048
_D = 128
_TOTAL = _B * _T
_PER_W = _TOTAL // _NW
_CHUNK = 128
_NCHUNK = _PER_W // _CHUNK


def _emb_kernel(ids_hbm, tok_hbm, pos_hbm, out_hbm, idx_v, buf_v, sem):
    wid = lax.axis_index("s") * _NC + lax.axis_index("c")
    base = wid * _PER_W
    t_base = lax.rem(base, _T)

    pltpu.sync_copy(ids_hbm.at[pl.ds(wid * _NCHUNK, _NCHUNK)], idx_v)

    pltpu.sync_copy(pos_hbm.at[pl.ds(t_base, _PER_W)], buf_v)

    copies = []
    for j in range(_NCHUNK):
        copies.append(
            pltpu.async_copy(
                tok_hbm.at[idx_v.at[j]],
                buf_v.at[pl.ds(j * _CHUNK, _CHUNK)],
                sem,
                add=True,
            )
        )
    for c in copies:
        c.wait()

    pltpu.sync_copy(buf_v, out_hbm.at[pl.ds(base, _PER_W)])


@jax.jit
def kernel(input_ids, token_table, pos_table):
    ids_flat = input_ids.reshape(_TOTAL // _CHUNK, _CHUNK).astype(jnp.int32)

    mesh = plsc.VectorSubcoreMesh(core_axis_name="c", subcore_axis_name="s")
    out = pl.kernel(
        _emb_kernel,
        out_type=jax.ShapeDtypeStruct((_TOTAL, _D), jnp.float32),
        mesh=mesh,
        scratch_types=[
            pltpu.VMEM((_NCHUNK, _CHUNK), jnp.int32),
            pltpu.VMEM((_PER_W, _D), jnp.float32),
            pltpu.SemaphoreType.DMA,
        ],
    )(ids_flat, token_table, pos_table)
    return out.reshape(_B, _T, _D)


# speedup vs baseline: 1.3822x; 1.0248x over previous
import jax
import jax.numpy as jnp
from jax import lax
from jax.experimental import pallas as pl
from jax.experimental.pallas import tpu as pltpu
from jax.experimental.pallas import tpu_sc as plsc

_NC = 2
_NS = 16
_NW = _NC * _NS

_B = 4
_T = 2048
_D = 128
_TOTAL = _B * _T
_PER_W = _TOTAL // _NW
_CHUNK = 128
_NCHUNK = _PER_W // _CHUNK


def _emb_kernel(ids_hbm, tok_hbm, pos_hbm, out_hbm, idx_v, buf_v,
                sem_p, sem_g, sem_w):
    wid = lax.axis_index("s") * _NC + lax.axis_index("c")
    base = wid * _PER_W
    t_base = lax.rem(base, _T)

    pos_copies = [
        pltpu.async_copy(
            pos_hbm.at[pl.ds(t_base + j * _CHUNK, _CHUNK)],
            buf_v.at[pl.ds(j * _CHUNK, _CHUNK)],
            sem_p[j],
        )
        for j in range(_NCHUNK)
    ]
    pltpu.sync_copy(ids_hbm.at[pl.ds(wid * _NCHUNK, _NCHUNK)], idx_v)

    gathers = []
    for j in range(_NCHUNK):
        pos_copies[j].wait()
        gathers.append(
            pltpu.async_copy(
                tok_hbm.at[idx_v.at[j]],
                buf_v.at[pl.ds(j * _CHUNK, _CHUNK)],
                sem_g[j],
                add=True,
            )
        )

    writes = []
    for j in range(_NCHUNK):
        gathers[j].wait()
        writes.append(
            pltpu.async_copy(
                buf_v.at[pl.ds(j * _CHUNK, _CHUNK)],
                out_hbm.at[pl.ds(base + j * _CHUNK, _CHUNK)],
                sem_w[j],
            )
        )
    for w in writes:
        w.wait()


@jax.jit
def kernel(input_ids, token_table, pos_table):
    ids_flat = input_ids.reshape(_TOTAL // _CHUNK, _CHUNK).astype(jnp.int32)

    mesh = plsc.VectorSubcoreMesh(core_axis_name="c", subcore_axis_name="s")
    out = pl.kernel(
        _emb_kernel,
        out_type=jax.ShapeDtypeStruct((_TOTAL, _D), jnp.float32),
        mesh=mesh,
        scratch_types=[
            pltpu.VMEM((_NCHUNK, _CHUNK), jnp.int32),
            pltpu.VMEM((_PER_W, _D), jnp.float32),
            [pltpu.SemaphoreType.DMA] * _NCHUNK,
            [pltpu.SemaphoreType.DMA] * _NCHUNK,
            [pltpu.SemaphoreType.DMA] * _NCHUNK,
        ],
    )(ids_flat, token_table, pos_table)
    return out.reshape(_B, _T, _D)
